# baseline (speedup 1.0000x reference)
import functools
import math

import jax
import jax.numpy as jnp
from jax import lax
from jax.experimental import pallas as pl
from jax.experimental.pallas import tpu as pltpu
from jax.experimental.pallas import tpu_sc as plsc

NUM_EMB = 100000
EMB_DIM = 64
BATCH = 16384

NC = 2
NS = 16
NW = NC * NS
B_PER_W = BATCH // NW
CHUNK = 128
NCHUNK = B_PER_W // CHUNK

_INV_SQRT2 = 1.0 / math.sqrt(2.0)
_Q = 25600
_J_BLK = 12800
_S_BLK = 8192


def _bf16_bits(act):
    bits = lax.bitcast_convert_type(act, jnp.uint32)
    return (bits + jnp.uint32(0x7FFF) + ((bits >> 16) & jnp.uint32(1))) >> 16


def _proj_body(a_ref, b_ref, c_ref, d_ref, w_ref, bias_ref, out_ref):
    projs = []
    for ref in (a_ref, b_ref, c_ref, d_ref):
        projs.append(
            lax.dot_general(ref[...], w_ref[...], (((0,), (1,)), ((), ())),
                            preferred_element_type=jnp.float32) + bias_ref[...])
    packed01 = _bf16_bits(projs[0]) | (_bf16_bits(projs[1]) << 16)
    packed23 = _bf16_bits(projs[2]) | (_bf16_bits(projs[3]) << 16)
    out_ref[:, :EMB_DIM] = packed01
    out_ref[:, EMB_DIM:] = packed23


def _tc_project_table(tt, w, b_col):
    nj = _Q // _J_BLK
    return pl.pallas_call(
        _proj_body,
        grid=(nj,),
        in_specs=[
            pl.BlockSpec((EMB_DIM, _J_BLK), lambda j, q=q, nj=nj: (0, j + q * nj))
            for q in range(4)
        ] + [
            pl.BlockSpec((EMB_DIM, EMB_DIM), lambda j: (0, 0)),
            pl.BlockSpec((1, EMB_DIM), lambda j: (0, 0)),
        ],
        out_specs=pl.BlockSpec((_J_BLK, 2 * EMB_DIM), lambda j: (j, 0)),
        out_shape=jax.ShapeDtypeStruct((_Q, 2 * EMB_DIM), jnp.uint32),
    )(tt, tt, tt, tt, w, b_col)


def _sc_gather_quads(p4, fold_ids):
    mesh = plsc.VectorSubcoreMesh(core_axis_name="c", subcore_axis_name="s")

    @functools.partial(
        pl.kernel,
        out_type=jax.ShapeDtypeStruct((BATCH, 2 * EMB_DIM), jnp.uint32),
        mesh=mesh,
        scratch_types=[
            pltpu.VMEM((B_PER_W,), jnp.int32),
            pltpu.VMEM((B_PER_W, 2 * EMB_DIM), jnp.uint32),
            pltpu.SemaphoreType.DMA,
        ],
        compiler_params=pltpu.CompilerParams(use_tc_tiling_on_sc=True),
    )
    def k(p4_hbm, idx_hbm, out_hbm, idx_v, rows_v, sem):
        wid = lax.axis_index("s") * NC + lax.axis_index("c")
        base = wid * B_PER_W
        pltpu.sync_copy(idx_hbm.at[pl.ds(base, B_PER_W)], idx_v)
        copies = []
        for j in range(NCHUNK):
            copies.append(
                pltpu.async_copy(
                    p4_hbm.at[idx_v.at[pl.ds(j * CHUNK, CHUNK)]],
                    rows_v.at[pl.ds(j * CHUNK, CHUNK)],
                    sem,
                )
            )
        for c in copies:
            c.wait()
        pltpu.sync_copy(rows_v, out_hbm.at[pl.ds(base, B_PER_W)])

    return k(p4, fold_ids)


def _sel_body(quads_ref, q_ref, eye_ref, out_ref):
    p = quads_ref[...]
    qc = lax.transpose(q_ref[...], (1, 0))
    u = jnp.where(qc >= 2, p[:, EMB_DIM:], p[:, :EMB_DIM])
    f32_bits = jnp.where((qc & 1) == 1,
                         u & jnp.uint32(0xFFFF0000),
                         u << 16)
    proj = lax.bitcast_convert_type(f32_bits, jnp.float32)
    act = 0.5 * proj * (1.0 + lax.erf(proj * _INV_SQRT2))
    out_ref[...] = lax.dot_general(eye_ref[...], act, (((1,), (1,)), ((), ())),
                                   preferred_element_type=jnp.float32)


def _tc_select_quarter(quads, q_row, eye):
    return pl.pallas_call(
        _sel_body,
        grid=(BATCH // _S_BLK,),
        in_specs=[
            pl.BlockSpec((_S_BLK, 2 * EMB_DIM), lambda i: (i, 0)),
            pl.BlockSpec((1, _S_BLK), lambda i: (0, i)),
            pl.BlockSpec((EMB_DIM, EMB_DIM), lambda i: (0, 0)),
        ],
        out_specs=pl.BlockSpec((EMB_DIM, _S_BLK), lambda i: (0, i)),
        out_shape=jax.ShapeDtypeStruct((EMB_DIM, BATCH), jnp.float32),
    )(quads, q_row, eye)


def kernel(country_ids, table, W, b):
    ids = country_ids.astype(jnp.int32)
    q = ids // _Q
    fold_ids = ids - q * _Q
    q_row = q.reshape(1, BATCH)
    p4 = _tc_project_table(table.T, W, b.reshape(1, EMB_DIM))
    quads = _sc_gather_quads(p4, fold_ids)
    eye = jnp.eye(EMB_DIM, dtype=jnp.float32)
    return _tc_select_quarter(quads, q_row, eye).T


# speedup vs baseline: 1.0476x; 1.0476x over previous
import functools
import math

import jax
import jax.numpy as jnp
from jax import lax
from jax.experimental import pallas as pl
from jax.experimental.pallas import tpu as pltpu
from jax.experimental.pallas import tpu_sc as plsc

NUM_EMB = 100000
EMB_DIM = 64
BATCH = 16384

NC = 2
NS = 16
NW = NC * NS
B_PER_W = BATCH // NW
CHUNK = 128
NCHUNK = B_PER_W // CHUNK

_INV_SQRT2 = 1.0 / math.sqrt(2.0)
_Q = 25600
_J_BLK = 6400
_S_BLK = 4096


def _bf16_bits(act):
    bits = lax.bitcast_convert_type(act, jnp.uint32)
    return (bits + jnp.uint32(0x7FFF) + ((bits >> 16) & jnp.uint32(1))) >> 16


def _proj_body(a_ref, b_ref, c_ref, d_ref, w_ref, bias_ref, out_ref):
    projs = []
    for ref in (a_ref, b_ref, c_ref, d_ref):
        projs.append(
            lax.dot_general(ref[...], w_ref[...], (((0,), (1,)), ((), ())),
                            preferred_element_type=jnp.float32) + bias_ref[...])
    packed01 = _bf16_bits(projs[0]) | (_bf16_bits(projs[1]) << 16)
    packed23 = _bf16_bits(projs[2]) | (_bf16_bits(projs[3]) << 16)
    out_ref[:, :EMB_DIM] = packed01
    out_ref[:, EMB_DIM:] = packed23


def _tc_project_table(tt, w, b_col):
    nj = _Q // _J_BLK
    return pl.pallas_call(
        _proj_body,
        grid=(nj,),
        in_specs=[
            pl.BlockSpec((EMB_DIM, _J_BLK), lambda j, q=q, nj=nj: (0, j + q * nj))
            for q in range(4)
        ] + [
            pl.BlockSpec((EMB_DIM, EMB_DIM), lambda j: (0, 0)),
            pl.BlockSpec((1, EMB_DIM), lambda j: (0, 0)),
        ],
        out_specs=pl.BlockSpec((_J_BLK, 2 * EMB_DIM), lambda j: (j, 0)),
        out_shape=jax.ShapeDtypeStruct((_Q, 2 * EMB_DIM), jnp.uint32),
    )(tt, tt, tt, tt, w, b_col)


def _sc_gather_quads(p4, fold_ids):
    mesh = plsc.VectorSubcoreMesh(core_axis_name="c", subcore_axis_name="s")

    @functools.partial(
        pl.kernel,
        out_type=jax.ShapeDtypeStruct((BATCH, 2 * EMB_DIM), jnp.uint32),
        mesh=mesh,
        scratch_types=[
            pltpu.VMEM((B_PER_W,), jnp.int32),
            pltpu.VMEM((B_PER_W, 2 * EMB_DIM), jnp.uint32),
            pltpu.SemaphoreType.DMA,
        ],
        compiler_params=pltpu.CompilerParams(use_tc_tiling_on_sc=True),
    )
    def k(p4_hbm, idx_hbm, out_hbm, idx_v, rows_v, sem):
        wid = lax.axis_index("s") * NC + lax.axis_index("c")
        base = wid * B_PER_W
        pltpu.sync_copy(idx_hbm.at[pl.ds(base, B_PER_W)], idx_v)
        copies = []
        for j in range(NCHUNK):
            copies.append(
                pltpu.async_copy(
                    p4_hbm.at[idx_v.at[pl.ds(j * CHUNK, CHUNK)]],
                    rows_v.at[pl.ds(j * CHUNK, CHUNK)],
                    sem,
                )
            )
        for c in copies:
            c.wait()
        pltpu.sync_copy(rows_v, out_hbm.at[pl.ds(base, B_PER_W)])

    return k(p4, fold_ids)


def _sel_body(quads_ref, q_ref, eye_ref, out_ref):
    p = quads_ref[...]
    qc = lax.transpose(q_ref[...], (1, 0))
    u = jnp.where(qc >= 2, p[:, EMB_DIM:], p[:, :EMB_DIM])
    f32_bits = jnp.where((qc & 1) == 1,
                         u & jnp.uint32(0xFFFF0000),
                         u << 16)
    proj = lax.bitcast_convert_type(f32_bits, jnp.float32)
    act = 0.5 * proj * (1.0 + lax.erf(proj * _INV_SQRT2))
    out_ref[...] = lax.dot_general(eye_ref[...], act, (((1,), (1,)), ((), ())),
                                   preferred_element_type=jnp.float32)


def _tc_select_quarter(quads, q_row, eye):
    return pl.pallas_call(
        _sel_body,
        grid=(BATCH // _S_BLK,),
        in_specs=[
            pl.BlockSpec((_S_BLK, 2 * EMB_DIM), lambda i: (i, 0)),
            pl.BlockSpec((1, _S_BLK), lambda i: (0, i)),
            pl.BlockSpec((EMB_DIM, EMB_DIM), lambda i: (0, 0)),
        ],
        out_specs=pl.BlockSpec((EMB_DIM, _S_BLK), lambda i: (0, i)),
        out_shape=jax.ShapeDtypeStruct((EMB_DIM, BATCH), jnp.float32),
    )(quads, q_row, eye)


def kernel(country_ids, table, W, b):
    ids = country_ids.astype(jnp.int32)
    q = ids // _Q
    fold_ids = ids - q * _Q
    q_row = q.reshape(1, BATCH)
    p4 = _tc_project_table(table.T, W, b.reshape(1, EMB_DIM))
    quads = _sc_gather_quads(p4, fold_ids)
    eye = jnp.eye(EMB_DIM, dtype=jnp.float32)
    return _tc_select_quarter(quads, q_row, eye).T


# speedup vs baseline: 1.0523x; 1.0045x over previous
import functools
import math

import jax
import jax.numpy as jnp
from jax import lax
from jax.experimental import pallas as pl
from jax.experimental.pallas import tpu as pltpu
from jax.experimental.pallas import tpu_sc as plsc

NUM_EMB = 100000
EMB_DIM = 64
BATCH = 16384

NC = 2
NS = 16
NW = NC * NS
B_PER_W = BATCH // NW
CHUNK = 128
NCHUNK = B_PER_W // CHUNK

_INV_SQRT2 = 1.0 / math.sqrt(2.0)
_Q = 25600
_J_BLK = 5120
_S_BLK = 4096


def _bf16_bits(act):
    bits = lax.bitcast_convert_type(act, jnp.uint32)
    return (bits + jnp.uint32(0x7FFF) + ((bits >> 16) & jnp.uint32(1))) >> 16


def _proj_body(a_ref, b_ref, c_ref, d_ref, w_ref, bias_ref, out_ref):
    projs = []
    for ref in (a_ref, b_ref, c_ref, d_ref):
        projs.append(
            lax.dot_general(ref[...], w_ref[...], (((0,), (1,)), ((), ())),
                            preferred_element_type=jnp.float32) + bias_ref[...])
    packed01 = _bf16_bits(projs[0]) | (_bf16_bits(projs[1]) << 16)
    packed23 = _bf16_bits(projs[2]) | (_bf16_bits(projs[3]) << 16)
    out_ref[:, :EMB_DIM] = packed01
    out_ref[:, EMB_DIM:] = packed23


def _tc_project_table(tt, w, b_col):
    nj = _Q // _J_BLK
    return pl.pallas_call(
        _proj_body,
        grid=(nj,),
        in_specs=[
            pl.BlockSpec((EMB_DIM, _J_BLK), lambda j, q=q, nj=nj: (0, j + q * nj))
            for q in range(4)
        ] + [
            pl.BlockSpec((EMB_DIM, EMB_DIM), lambda j: (0, 0)),
            pl.BlockSpec((1, EMB_DIM), lambda j: (0, 0)),
        ],
        out_specs=pl.BlockSpec((_J_BLK, 2 * EMB_DIM), lambda j: (j, 0)),
        out_shape=jax.ShapeDtypeStruct((_Q, 2 * EMB_DIM), jnp.uint32),
    )(tt, tt, tt, tt, w, b_col)


def _sc_gather_quads(p4, fold_ids):
    mesh = plsc.VectorSubcoreMesh(core_axis_name="c", subcore_axis_name="s")

    @functools.partial(
        pl.kernel,
        out_type=jax.ShapeDtypeStruct((BATCH, 2 * EMB_DIM), jnp.uint32),
        mesh=mesh,
        scratch_types=[
            pltpu.VMEM((B_PER_W,), jnp.int32),
            pltpu.VMEM((B_PER_W, 2 * EMB_DIM), jnp.uint32),
            pltpu.SemaphoreType.DMA,
        ],
        compiler_params=pltpu.CompilerParams(use_tc_tiling_on_sc=True),
    )
    def k(p4_hbm, idx_hbm, out_hbm, idx_v, rows_v, sem):
        wid = lax.axis_index("s") * NC + lax.axis_index("c")
        base = wid * B_PER_W
        pltpu.sync_copy(idx_hbm.at[pl.ds(base, B_PER_W)], idx_v)
        copies = []
        for j in range(NCHUNK):
            copies.append(
                pltpu.async_copy(
                    p4_hbm.at[idx_v.at[pl.ds(j * CHUNK, CHUNK)]],
                    rows_v.at[pl.ds(j * CHUNK, CHUNK)],
                    sem,
                )
            )
        for c in copies:
            c.wait()
        pltpu.sync_copy(rows_v, out_hbm.at[pl.ds(base, B_PER_W)])

    return k(p4, fold_ids)


def _sel_body(quads_ref, q_ref, eye_ref, out_ref):
    p = quads_ref[...]
    qc = lax.transpose(q_ref[...], (1, 0))
    u = jnp.where(qc >= 2, p[:, EMB_DIM:], p[:, :EMB_DIM])
    f32_bits = jnp.where((qc & 1) == 1,
                         u & jnp.uint32(0xFFFF0000),
                         u << 16)
    proj = lax.bitcast_convert_type(f32_bits, jnp.float32)
    act = 0.5 * proj * (1.0 + lax.erf(proj * _INV_SQRT2))
    out_ref[...] = lax.dot_general(eye_ref[...], act, (((1,), (1,)), ((), ())),
                                   preferred_element_type=jnp.float32)


def _tc_select_quarter(quads, q_row, eye):
    return pl.pallas_call(
        _sel_body,
        grid=(BATCH // _S_BLK,),
        in_specs=[
            pl.BlockSpec((_S_BLK, 2 * EMB_DIM), lambda i: (i, 0)),
            pl.BlockSpec((1, _S_BLK), lambda i: (0, i)),
            pl.BlockSpec((EMB_DIM, EMB_DIM), lambda i: (0, 0)),
        ],
        out_specs=pl.BlockSpec((EMB_DIM, _S_BLK), lambda i: (0, i)),
        out_shape=jax.ShapeDtypeStruct((EMB_DIM, BATCH), jnp.float32),
    )(quads, q_row, eye)


def kernel(country_ids, table, W, b):
    ids = country_ids.astype(jnp.int32)
    q = ids // _Q
    fold_ids = ids - q * _Q
    q_row = q.reshape(1, BATCH)
    p4 = _tc_project_table(table.T, W, b.reshape(1, EMB_DIM))
    quads = _sc_gather_quads(p4, fold_ids)
    eye = jnp.eye(EMB_DIM, dtype=jnp.float32)
    return _tc_select_quarter(quads, q_row, eye).T
